# baseline (speedup 1.0000x reference)
import functools

import jax
import jax.numpy as jnp
from jax import lax
from jax.experimental import pallas as pl
from jax.experimental.pallas import tpu as pltpu
from jax.experimental.pallas import tpu_sc as plsc

_NC = 2
_NS = 16
_NW = _NC * _NS
_L = 16

_IN = 256
_H1 = 8192
_H2 = 1024
_OUT = 1000

_SCR = 2048
_TCR = _H1 - _SCR
_R1 = _SCR // _NW
_CH1 = 32
_NCH1 = _R1 // _CH1

_BIG_F = 3.0e38
_BIG_I = 2**30


def _wid():
    return lax.axis_index("s") * _NC + lax.axis_index("c")


def _mesh():
    return plsc.VectorSubcoreMesh(core_axis_name="c", subcore_axis_name="s")


def _iota():
    return lax.iota(jnp.int32, _L)


def _merge_candidates(vals_ref, idx_ref):
    runv = vals_ref[0, :]
    for w in range(1, _NW):
        runv = jnp.minimum(runv, vals_ref[w, :])
    mv = jnp.min(runv)
    runi = jnp.full((_L,), _BIG_I, jnp.int32)
    for w in range(_NW):
        runi = jnp.minimum(
            runi, jnp.where(vals_ref[w, :] == mv, idx_ref[w, :], _BIG_I))
    return jnp.min(runi)


@functools.partial(
    pl.kernel,
    out_type=(
        jax.ShapeDtypeStruct((_NW, _L), jnp.float32),
        jax.ShapeDtypeStruct((_NW, _L), jnp.int32),
    ),
    mesh=_mesh(),
    compiler_params=pltpu.CompilerParams(needs_layout_passes=False),
    scratch_types=[
        pltpu.VMEM((2, _CH1, _IN), jnp.float32),
        pltpu.VMEM((1, _IN), jnp.float32),
        pltpu.VMEM((_L,), jnp.float32),
        pltpu.VMEM((_L,), jnp.int32),
        pltpu.SemaphoreType.DMA,
        pltpu.SemaphoreType.DMA,
    ],
)
def _ka(w1_hbm, x_hbm, vals_hbm, idx_hbm, w1_v, x_v, cv_v, ci_v, sem0, sem1):
    wid = _wid()
    iota = _iota()
    base1 = _TCR + wid * _R1
    sems = (sem0, sem1)

    for b in range(2):
        pltpu.make_async_copy(
            w1_hbm.at[pl.ds(base1 + b * _CH1, _CH1)],
            w1_v.at[b], sems[b]).start()
    pltpu.sync_copy(x_hbm, x_v)

    ngroups = _CH1 // _L
    runv = jnp.full((_L,), _BIG_F, jnp.float32)
    runi = jnp.zeros((_L,), jnp.int32)
    for cc in range(_NCH1):
        b = cc % 2
        pltpu.make_async_copy(
            w1_hbm.at[pl.ds(base1 + cc * _CH1, _CH1)],
            w1_v.at[b], sems[b]).wait()

        def body(k2, accs, b=b):
            new = list(accs)
            for u in range(2):
                k = k2 * 2 + u
                kv = jnp.full((_L,), k, jnp.int32)
                xb = plsc.load_gather(
                    x_v, [jnp.zeros((_L,), jnp.int32), kv])
                for g in range(ngroups):
                    w = plsc.load_gather(
                        w1_v, [jnp.full((_L,), b, jnp.int32),
                               g * _L + iota, kv])
                    d = w - xb
                    new[g] = new[g] + d * d
            return tuple(new)

        accs = lax.fori_loop(
            0, _IN // 2, body,
            tuple(jnp.zeros((_L,), jnp.float32) for _ in range(ngroups)))

        for g in range(ngroups):
            gidx = base1 + cc * _CH1 + g * _L + iota
            sel = accs[g] < runv
            runv = jnp.where(sel, accs[g], runv)
            runi = jnp.where(sel, gidx, runi)

        nxt = cc + 2
        if nxt < _NCH1:
            pltpu.make_async_copy(
                w1_hbm.at[pl.ds(base1 + nxt * _CH1, _CH1)],
                w1_v.at[b], sems[b]).start()

    cv_v[...] = runv
    ci_v[...] = runi
    pltpu.sync_copy(cv_v, vals_hbm.at[wid])
    pltpu.sync_copy(ci_v, idx_hbm.at[wid])


_TCB = 256
_NB = _H2 // _TCB
_TB1 = _TCR // _NB


def _kb_body(w_ref, w1_ref, x_ref, o_ref, bm_ref, bi_ref):
    w = w_ref[...]
    o_ref[...] = jnp.sum(w * w, axis=1, keepdims=True)
    step = pl.program_id(0)
    d1 = w1_ref[...] - x_ref[...]
    d2 = jnp.sum(d1 * d1, axis=1, keepdims=True)
    bm = jnp.min(d2)
    rows = lax.broadcasted_iota(jnp.int32, (_TB1, 1), 0) + step * _TB1
    bm_ref[0, 0, 0] = bm
    bi_ref[0, 0, 0] = jnp.min(jnp.where(d2 == bm, rows, _BIG_I))


_kb = pl.pallas_call(
    _kb_body,
    grid=(_NB,),
    in_specs=[
        pl.BlockSpec((_TCB, _H1), lambda i: (i, 0)),
        pl.BlockSpec((_TB1, _IN), lambda i: (i, 0)),
        pl.BlockSpec((1, _IN), lambda i: (0, 0)),
    ],
    out_specs=[
        pl.BlockSpec((_TCB, 1), lambda i: (i, 0)),
        pl.BlockSpec((1, 1, 1), lambda i: (i, 0, 0), memory_space=pltpu.SMEM),
        pl.BlockSpec((1, 1, 1), lambda i: (i, 0, 0), memory_space=pltpu.SMEM),
    ],
    out_shape=[
        jax.ShapeDtypeStruct((_H2, 1), jnp.float32),
        jax.ShapeDtypeStruct((_NB, 1, 1), jnp.float32),
        jax.ShapeDtypeStruct((_NB, 1, 1), jnp.int32),
    ],
)


def _kc_body(v1_ref, i1_ref, bm_ref, bi_ref, n2_ref, w2_ref, fcw_ref,
             fcb_ref, out_ref, w2c, fcc, sem0, sem1):
    bm = bm_ref[:, :, 0]
    mv_tc = jnp.min(bm)
    i_tc = jnp.min(jnp.where(bm == mv_tc, bi_ref[:, :, 0], _BIG_I))
    v1 = v1_ref[...]
    i1 = i1_ref[...]
    mv_sc = jnp.min(v1)
    i_sc = jnp.min(jnp.where(v1 == mv_sc, i1, _BIG_I))
    c1 = jnp.where(mv_tc <= mv_sc, i_tc, i_sc)
    c1a = pl.multiple_of((c1 // 128) * 128, 128)
    c1m = c1 % 128
    cp1 = pltpu.make_async_copy(w2_ref.at[:, pl.ds(c1a, 128)], w2c, sem0)
    cp1.start()
    cp1.wait()
    lanes = lax.broadcasted_iota(jnp.int32, (_H2, 128), 1)
    sm = jnp.where(lanes == c1m, n2_ref[...] - 2.0 * w2c[...], _BIG_F)
    mv2 = jnp.min(sm)
    rows = lax.broadcasted_iota(jnp.int32, (_H2, 128), 0)
    c2 = jnp.min(jnp.where(sm == mv2, rows, _BIG_I))
    c2a = pl.multiple_of((c2 // 128) * 128, 128)
    c2m = c2 % 128
    cp2 = pltpu.make_async_copy(fcw_ref.at[:, pl.ds(c2a, 128)], fcc, sem1)
    cp2.start()
    cp2.wait()
    lanes2 = lax.broadcasted_iota(jnp.int32, (_OUT, 128), 1)
    fcol = jnp.sum(jnp.where(lanes2 == c2m, fcc[...], 0.0), axis=1)
    out_ref[...] = fcol + fcb_ref[...]


_kc = pl.pallas_call(
    _kc_body,
    in_specs=[
        pl.BlockSpec(memory_space=pltpu.VMEM),
        pl.BlockSpec(memory_space=pltpu.VMEM),
        pl.BlockSpec(memory_space=pltpu.VMEM),
        pl.BlockSpec(memory_space=pltpu.VMEM),
        pl.BlockSpec(memory_space=pltpu.VMEM),
        pl.BlockSpec(memory_space=pl.ANY),
        pl.BlockSpec(memory_space=pl.ANY),
        pl.BlockSpec(memory_space=pltpu.VMEM),
    ],
    out_specs=pl.BlockSpec(memory_space=pltpu.VMEM),
    out_shape=jax.ShapeDtypeStruct((_OUT,), jnp.float32),
    scratch_shapes=[
        pltpu.VMEM((_H2, 128), jnp.float32),
        pltpu.VMEM((_OUT, 128), jnp.float32),
        pltpu.SemaphoreType.DMA,
        pltpu.SemaphoreType.DMA,
    ],
)


def kernel(x, reward, W1, T1, W2, T2, fc_w, fc_b):
    del reward, T1, T2
    x2 = x.reshape(1, _IN)
    n2, bm, bi = _kb(W2, W1, x2)
    v1, i1 = _ka(W1, x2)
    return _kc(v1, i1, bm, bi, n2, W2, fc_w, fc_b)


# speedup vs baseline: 1.0002x; 1.0002x over previous
import functools

import jax
import jax.numpy as jnp
from jax import lax
from jax.experimental import pallas as pl
from jax.experimental.pallas import tpu as pltpu
from jax.experimental.pallas import tpu_sc as plsc

_NC = 2
_NS = 16
_NW = _NC * _NS
_L = 16

_IN = 256
_H1 = 8192
_H2 = 1024
_OUT = 1000

_SCR = 2048
_TCR = _H1 - _SCR
_R1 = _SCR // _NW
_CH1 = 32
_NCH1 = _R1 // _CH1

_BIG_F = 3.0e38
_BIG_I = 2**30


def _wid():
    return lax.axis_index("s") * _NC + lax.axis_index("c")


def _mesh():
    return plsc.VectorSubcoreMesh(core_axis_name="c", subcore_axis_name="s")


def _iota():
    return lax.iota(jnp.int32, _L)


def _merge_candidates(vals_ref, idx_ref):
    runv = vals_ref[0, :]
    for w in range(1, _NW):
        runv = jnp.minimum(runv, vals_ref[w, :])
    mv = jnp.min(runv)
    runi = jnp.full((_L,), _BIG_I, jnp.int32)
    for w in range(_NW):
        runi = jnp.minimum(
            runi, jnp.where(vals_ref[w, :] == mv, idx_ref[w, :], _BIG_I))
    return jnp.min(runi)


@functools.partial(
    pl.kernel,
    out_type=(
        jax.ShapeDtypeStruct((_NW, _L), jnp.float32),
        jax.ShapeDtypeStruct((_NW, _L), jnp.int32),
    ),
    mesh=_mesh(),
    compiler_params=pltpu.CompilerParams(needs_layout_passes=False),
    scratch_types=[
        pltpu.VMEM((2, _CH1, _IN), jnp.float32),
        pltpu.VMEM((1, _IN), jnp.float32),
        pltpu.VMEM((_L,), jnp.float32),
        pltpu.VMEM((_L,), jnp.int32),
        pltpu.SemaphoreType.DMA,
        pltpu.SemaphoreType.DMA,
    ],
)
def _ka(w1_hbm, x_hbm, vals_hbm, idx_hbm, w1_v, x_v, cv_v, ci_v, sem0, sem1):
    wid = _wid()
    iota = _iota()
    base1 = _TCR + wid * _R1
    sems = (sem0, sem1)

    for b in range(2):
        pltpu.make_async_copy(
            w1_hbm.at[pl.ds(base1 + b * _CH1, _CH1)],
            w1_v.at[b], sems[b]).start()
    pltpu.sync_copy(x_hbm, x_v)

    ngroups = _CH1 // _L
    runv = jnp.full((_L,), _BIG_F, jnp.float32)
    runi = jnp.zeros((_L,), jnp.int32)
    for cc in range(_NCH1):
        b = cc % 2
        pltpu.make_async_copy(
            w1_hbm.at[pl.ds(base1 + cc * _CH1, _CH1)],
            w1_v.at[b], sems[b]).wait()

        def body(k2, accs, b=b):
            new = list(accs)
            for u in range(2):
                k = k2 * 2 + u
                kv = jnp.full((_L,), k, jnp.int32)
                xb = plsc.load_gather(
                    x_v, [jnp.zeros((_L,), jnp.int32), kv])
                for g in range(ngroups):
                    w = plsc.load_gather(
                        w1_v, [jnp.full((_L,), b, jnp.int32),
                               g * _L + iota, kv])
                    d = w - xb
                    new[g] = new[g] + d * d
            return tuple(new)

        accs = lax.fori_loop(
            0, _IN // 2, body,
            tuple(jnp.zeros((_L,), jnp.float32) for _ in range(ngroups)))

        for g in range(ngroups):
            gidx = base1 + cc * _CH1 + g * _L + iota
            sel = accs[g] < runv
            runv = jnp.where(sel, accs[g], runv)
            runi = jnp.where(sel, gidx, runi)

        nxt = cc + 2
        if nxt < _NCH1:
            pltpu.make_async_copy(
                w1_hbm.at[pl.ds(base1 + nxt * _CH1, _CH1)],
                w1_v.at[b], sems[b]).start()

    cv_v[...] = runv
    ci_v[...] = runi
    pltpu.sync_copy(cv_v, vals_hbm.at[wid])
    pltpu.sync_copy(ci_v, idx_hbm.at[wid])


_TCB = 128
_NB = _H2 // _TCB
_TB1 = _TCR // _NB


def _kb_body(w_ref, w1_ref, x_ref, o_ref, bm_ref, bi_ref):
    w = w_ref[...]
    o_ref[...] = jnp.sum(w * w, axis=1, keepdims=True)
    step = pl.program_id(0)
    d1 = w1_ref[...] - x_ref[...]
    d2 = jnp.sum(d1 * d1, axis=1, keepdims=True)
    bm = jnp.min(d2)
    rows = lax.broadcasted_iota(jnp.int32, (_TB1, 1), 0) + step * _TB1
    bm_ref[0, 0, 0] = bm
    bi_ref[0, 0, 0] = jnp.min(jnp.where(d2 == bm, rows, _BIG_I))


_kb = pl.pallas_call(
    _kb_body,
    grid=(_NB,),
    in_specs=[
        pl.BlockSpec((_TCB, _H1), lambda i: (i, 0)),
        pl.BlockSpec((_TB1, _IN), lambda i: (i, 0)),
        pl.BlockSpec((1, _IN), lambda i: (0, 0)),
    ],
    out_specs=[
        pl.BlockSpec((_TCB, 1), lambda i: (i, 0)),
        pl.BlockSpec((1, 1, 1), lambda i: (i, 0, 0), memory_space=pltpu.SMEM),
        pl.BlockSpec((1, 1, 1), lambda i: (i, 0, 0), memory_space=pltpu.SMEM),
    ],
    out_shape=[
        jax.ShapeDtypeStruct((_H2, 1), jnp.float32),
        jax.ShapeDtypeStruct((_NB, 1, 1), jnp.float32),
        jax.ShapeDtypeStruct((_NB, 1, 1), jnp.int32),
    ],
)


def _kc_body(v1_ref, i1_ref, bm_ref, bi_ref, n2_ref, w2_ref, fcw_ref,
             fcb_ref, out_ref, w2c, fcc, sem0, sem1):
    bm = bm_ref[:, :, 0]
    mv_tc = jnp.min(bm)
    i_tc = jnp.min(jnp.where(bm == mv_tc, bi_ref[:, :, 0], _BIG_I))
    v1 = v1_ref[...]
    i1 = i1_ref[...]
    mv_sc = jnp.min(v1)
    i_sc = jnp.min(jnp.where(v1 == mv_sc, i1, _BIG_I))
    c1 = jnp.where(mv_tc <= mv_sc, i_tc, i_sc)
    c1a = pl.multiple_of((c1 // 128) * 128, 128)
    c1m = c1 % 128
    cp1 = pltpu.make_async_copy(w2_ref.at[:, pl.ds(c1a, 128)], w2c, sem0)
    cp1.start()
    cp1.wait()
    lanes = lax.broadcasted_iota(jnp.int32, (_H2, 128), 1)
    sm = jnp.where(lanes == c1m, n2_ref[...] - 2.0 * w2c[...], _BIG_F)
    mv2 = jnp.min(sm)
    rows = lax.broadcasted_iota(jnp.int32, (_H2, 128), 0)
    c2 = jnp.min(jnp.where(sm == mv2, rows, _BIG_I))
    c2a = pl.multiple_of((c2 // 128) * 128, 128)
    c2m = c2 % 128
    cp2 = pltpu.make_async_copy(fcw_ref.at[:, pl.ds(c2a, 128)], fcc, sem1)
    cp2.start()
    cp2.wait()
    lanes2 = lax.broadcasted_iota(jnp.int32, (_OUT, 128), 1)
    fcol = jnp.sum(jnp.where(lanes2 == c2m, fcc[...], 0.0), axis=1)
    out_ref[...] = fcol + fcb_ref[...]


_kc = pl.pallas_call(
    _kc_body,
    in_specs=[
        pl.BlockSpec(memory_space=pltpu.VMEM),
        pl.BlockSpec(memory_space=pltpu.VMEM),
        pl.BlockSpec(memory_space=pltpu.VMEM),
        pl.BlockSpec(memory_space=pltpu.VMEM),
        pl.BlockSpec(memory_space=pltpu.VMEM),
        pl.BlockSpec(memory_space=pl.ANY),
        pl.BlockSpec(memory_space=pl.ANY),
        pl.BlockSpec(memory_space=pltpu.VMEM),
    ],
    out_specs=pl.BlockSpec(memory_space=pltpu.VMEM),
    out_shape=jax.ShapeDtypeStruct((_OUT,), jnp.float32),
    scratch_shapes=[
        pltpu.VMEM((_H2, 128), jnp.float32),
        pltpu.VMEM((_OUT, 128), jnp.float32),
        pltpu.SemaphoreType.DMA,
        pltpu.SemaphoreType.DMA,
    ],
)


def kernel(x, reward, W1, T1, W2, T2, fc_w, fc_b):
    del reward, T1, T2
    x2 = x.reshape(1, _IN)
    n2, bm, bi = _kb(W2, W1, x2)
    v1, i1 = _ka(W1, x2)
    return _kc(v1, i1, bm, bi, n2, W2, fc_w, fc_b)


# speedup vs baseline: 1.0081x; 1.0078x over previous
import functools

import jax
import jax.numpy as jnp
from jax import lax
from jax.experimental import pallas as pl
from jax.experimental.pallas import tpu as pltpu
from jax.experimental.pallas import tpu_sc as plsc

_NC = 2
_NS = 16
_NW = _NC * _NS
_L = 16

_IN = 256
_H1 = 8192
_H2 = 1024
_OUT = 1000

_SCR = 2048
_TCR = _H1 - _SCR
_R1 = _SCR // _NW
_CH1 = 32
_NCH1 = _R1 // _CH1

_BIG_F = 3.0e38
_BIG_I = 2**30


def _wid():
    return lax.axis_index("s") * _NC + lax.axis_index("c")


def _mesh():
    return plsc.VectorSubcoreMesh(core_axis_name="c", subcore_axis_name="s")


def _iota():
    return lax.iota(jnp.int32, _L)


@functools.partial(
    pl.kernel,
    out_type=(
        jax.ShapeDtypeStruct((_NW, _L), jnp.float32),
        jax.ShapeDtypeStruct((_NW, _L), jnp.int32),
    ),
    mesh=_mesh(),
    compiler_params=pltpu.CompilerParams(needs_layout_passes=False),
    scratch_types=[
        pltpu.VMEM((2, _CH1, _IN), jnp.float32),
        pltpu.VMEM((1, _IN), jnp.float32),
        pltpu.VMEM((_L,), jnp.float32),
        pltpu.VMEM((_L,), jnp.int32),
        pltpu.SemaphoreType.DMA,
        pltpu.SemaphoreType.DMA,
    ],
)
def _ka(w1_hbm, x_hbm, vals_hbm, idx_hbm, w1_v, x_v, cv_v, ci_v, sem0, sem1):
    wid = _wid()
    iota = _iota()
    base1 = _TCR + wid * _R1
    sems = (sem0, sem1)

    for b in range(2):
        pltpu.make_async_copy(
            w1_hbm.at[pl.ds(base1 + b * _CH1, _CH1)],
            w1_v.at[b], sems[b]).start()
    pltpu.sync_copy(x_hbm, x_v)

    ngroups = _CH1 // _L
    runv = jnp.full((_L,), _BIG_F, jnp.float32)
    runi = jnp.zeros((_L,), jnp.int32)
    for cc in range(_NCH1):
        b = cc % 2
        pltpu.make_async_copy(
            w1_hbm.at[pl.ds(base1 + cc * _CH1, _CH1)],
            w1_v.at[b], sems[b]).wait()

        def body(k2, accs, b=b):
            new = list(accs)
            for u in range(2):
                k = k2 * 2 + u
                kv = jnp.full((_L,), k, jnp.int32)
                xb = plsc.load_gather(
                    x_v, [jnp.zeros((_L,), jnp.int32), kv])
                for g in range(ngroups):
                    w = plsc.load_gather(
                        w1_v, [jnp.full((_L,), b, jnp.int32),
                               g * _L + iota, kv])
                    d = w - xb
                    new[g] = new[g] + d * d
            return tuple(new)

        accs = lax.fori_loop(
            0, _IN // 2, body,
            tuple(jnp.zeros((_L,), jnp.float32) for _ in range(ngroups)))

        for g in range(ngroups):
            gidx = base1 + cc * _CH1 + g * _L + iota
            sel = accs[g] < runv
            runv = jnp.where(sel, accs[g], runv)
            runi = jnp.where(sel, gidx, runi)

        nxt = cc + 2
        if nxt < _NCH1:
            pltpu.make_async_copy(
                w1_hbm.at[pl.ds(base1 + nxt * _CH1, _CH1)],
                w1_v.at[b], sems[b]).start()

    cv_v[...] = runv
    ci_v[...] = runi
    pltpu.sync_copy(cv_v, vals_hbm.at[wid])
    pltpu.sync_copy(ci_v, idx_hbm.at[wid])


_TCB = 128
_NB = _H2 // _TCB
_TB1 = _TCR // _NB


def _kb_body(w_ref, w1_ref, x_ref, o_ref, bm_ref, bi_ref):
    w = w_ref[...]
    o_ref[...] = jnp.sum(w * w, axis=1, keepdims=True)
    step = pl.program_id(0)
    d1 = w1_ref[...] - x_ref[...]
    d2 = jnp.sum(d1 * d1, axis=1, keepdims=True)
    bm = jnp.min(d2)
    rows = lax.broadcasted_iota(jnp.int32, (_TB1, 1), 0) + step * _TB1
    bm_ref[0, 0, 0] = bm
    bi_ref[0, 0, 0] = jnp.min(jnp.where(d2 == bm, rows, _BIG_I))


_kb = pl.pallas_call(
    _kb_body,
    grid=(_NB,),
    in_specs=[
        pl.BlockSpec((_TCB, _H1), lambda i: (i, 0)),
        pl.BlockSpec((_TB1, _IN), lambda i: (i, 0)),
        pl.BlockSpec((1, _IN), lambda i: (0, 0)),
    ],
    out_specs=[
        pl.BlockSpec((_TCB, 1), lambda i: (i, 0)),
        pl.BlockSpec((1, 1, 1), lambda i: (i, 0, 0), memory_space=pltpu.SMEM),
        pl.BlockSpec((1, 1, 1), lambda i: (i, 0, 0), memory_space=pltpu.SMEM),
    ],
    out_shape=[
        jax.ShapeDtypeStruct((_H2, 1), jnp.float32),
        jax.ShapeDtypeStruct((_NB, 1, 1), jnp.float32),
        jax.ShapeDtypeStruct((_NB, 1, 1), jnp.int32),
    ],
)


def _kc_body(v1_ref, i1_ref, bm_ref, bi_ref, n2_ref, w2_ref, fcw_ref,
             fcb_ref, out_ref, w2c, fcc, sem0, sem1):
    bm = bm_ref[:, :, 0]
    mv_tc = jnp.min(bm)
    i_tc = jnp.min(jnp.where(bm == mv_tc, bi_ref[:, :, 0], _BIG_I))
    v1 = v1_ref[...]
    i1 = i1_ref[...]
    mv_sc = jnp.min(v1)
    i_sc = jnp.min(jnp.where(v1 == mv_sc, i1, _BIG_I))
    c1 = jnp.where(mv_tc <= mv_sc, i_tc, i_sc)
    c1a = pl.multiple_of((c1 // 128) * 128, 128)
    c1m = c1 % 128
    cp1 = pltpu.make_async_copy(w2_ref.at[:, pl.ds(c1a, 128)], w2c, sem0)
    cp1.start()
    cp1.wait()
    lanes = lax.broadcasted_iota(jnp.int32, (_H2, 128), 1)
    sm = jnp.where(lanes == c1m, n2_ref[...] - 2.0 * w2c[...], _BIG_F)
    mv2 = jnp.min(sm)
    rows = lax.broadcasted_iota(jnp.int32, (_H2, 128), 0)
    c2 = jnp.min(jnp.where(sm == mv2, rows, _BIG_I))
    c2a = pl.multiple_of((c2 // 128) * 128, 128)
    c2m = c2 % 128
    cp2 = pltpu.make_async_copy(fcw_ref.at[:, pl.ds(c2a, 128)], fcc, sem1)
    cp2.start()
    cp2.wait()
    lanes2 = lax.broadcasted_iota(jnp.int32, (_OUT, 128), 1)
    fcol = jnp.sum(jnp.where(lanes2 == c2m, fcc[...], 0.0), axis=1)
    out_ref[...] = fcol + fcb_ref[...]


_kc = pl.pallas_call(
    _kc_body,
    in_specs=[
        pl.BlockSpec(memory_space=pltpu.VMEM),
        pl.BlockSpec(memory_space=pltpu.VMEM),
        pl.BlockSpec(memory_space=pltpu.VMEM),
        pl.BlockSpec(memory_space=pltpu.VMEM),
        pl.BlockSpec(memory_space=pltpu.VMEM),
        pl.BlockSpec(memory_space=pl.ANY),
        pl.BlockSpec(memory_space=pl.ANY),
        pl.BlockSpec(memory_space=pltpu.VMEM),
    ],
    out_specs=pl.BlockSpec(memory_space=pltpu.VMEM),
    out_shape=jax.ShapeDtypeStruct((_OUT,), jnp.float32),
    scratch_shapes=[
        pltpu.VMEM((_H2, 128), jnp.float32),
        pltpu.VMEM((_OUT, 128), jnp.float32),
        pltpu.SemaphoreType.DMA,
        pltpu.SemaphoreType.DMA,
    ],
)


def kernel(x, reward, W1, T1, W2, T2, fc_w, fc_b):
    del reward, T1, T2
    x2 = x.reshape(1, _IN)
    n2, bm, bi = _kb(W2, W1, x2)
    v1, i1 = _ka(W1, x2)
    return _kc(v1, i1, bm, bi, n2, W2, fc_w, fc_b)
